# baseline (speedup 1.0000x reference)
import functools

import jax
import jax.numpy as jnp
from jax import lax
from jax.experimental import pallas as pl
from jax.experimental.pallas import tpu as pltpu
from jax.experimental.pallas import tpu_sc as plsc

N = 10000
E = 320000
IN_DIM = 128
HID_DIM = 1024
OUT_DIM = 40
OUT_PAD = 48

NC, NS, LANES = 2, 16, 16
NW = NC * NS
CHUNK = 128
NCH = 80
NBUF = 2
COL = 64
E_PAD = NW * NCH * CHUNK
N_PAD = 10240
STRIPE = N_PAD // NS
PAD_ROW = N
DEG_W = 8

_vmesh = plsc.VectorSubcoreMesh(core_axis_name="c", subcore_axis_name="s")
_sc_params = pltpu.CompilerParams(use_tc_tiling_on_sc=False)


def _deg_body(dst_hbm, zeros_hbm, ones_hbm, out_hbm, idx_v, ones_v, deg_sh):
    cid = lax.axis_index("c")
    sid = lax.axis_index("s")
    wid = sid * NC + cid
    row0 = sid * STRIPE
    pltpu.sync_copy(zeros_hbm.at[pl.ds(row0, STRIPE)],
                    deg_sh.at[pl.ds(row0, STRIPE)])
    pltpu.sync_copy(dst_hbm.at[pl.ds(wid * NCH, NCH)], idx_v)
    pltpu.sync_copy(ones_hbm, ones_v)
    plsc.subcore_barrier()

    @pl.loop(0, NCH)
    def _(j):
        pltpu.sync_copy(ones_v, deg_sh.at[idx_v.at[j]], add=True)

    plsc.subcore_barrier()
    pltpu.sync_copy(deg_sh.at[pl.ds(row0, STRIPE)],
                    out_hbm.at[cid, pl.ds(row0, STRIPE)])


def _agg_body(y_hbm, src_hbm, dst_hbm, zeros_hbm, out_hbm,
              idxs_v, idxd_v, rows_v, y_sh, z_sh, sems):
    cid = lax.axis_index("c")
    sid = lax.axis_index("s")
    wid = sid * NC + cid
    row0 = sid * STRIPE
    pltpu.sync_copy(y_hbm.at[pl.ds(row0, STRIPE)], y_sh.at[pl.ds(row0, STRIPE)])
    pltpu.sync_copy(zeros_hbm.at[pl.ds(row0, STRIPE)],
                    z_sh.at[pl.ds(row0, STRIPE)])
    pltpu.sync_copy(src_hbm.at[pl.ds(wid * NCH, NCH)], idxs_v)
    pltpu.sync_copy(dst_hbm.at[pl.ds(wid * NCH, NCH)], idxd_v)
    plsc.subcore_barrier()

    @pl.loop(0, NBUF)
    def _(b):
        pltpu.async_copy(y_sh.at[idxs_v.at[b]], rows_v.at[b], sems.at[b])

    @pl.loop(0, NCH - NBUF)
    def _(j):
        b = lax.rem(j, NBUF)
        pltpu.make_async_copy(y_sh.at[idxs_v.at[0]], rows_v.at[b],
                              sems.at[b]).wait()
        pltpu.sync_copy(rows_v.at[b], z_sh.at[idxd_v.at[j]], add=True)
        pltpu.async_copy(y_sh.at[idxs_v.at[j + NBUF]], rows_v.at[b], sems.at[b])

    @pl.loop(NCH - NBUF, NCH)
    def _(j):
        b = lax.rem(j, NBUF)
        pltpu.make_async_copy(y_sh.at[idxs_v.at[0]], rows_v.at[b],
                              sems.at[b]).wait()
        pltpu.sync_copy(rows_v.at[b], z_sh.at[idxd_v.at[j]], add=True)

    plsc.subcore_barrier()
    pltpu.sync_copy(z_sh.at[pl.ds(row0, STRIPE)],
                    out_hbm.at[cid, pl.ds(row0, STRIPE)])


def _sc_degree(dst2d, zeros16, ones8):
    return pl.kernel(
        _deg_body,
        out_type=jax.ShapeDtypeStruct((NC, N_PAD, DEG_W), jnp.float32),
        mesh=_vmesh,
        scratch_types=[
            pltpu.VMEM((NCH, CHUNK), jnp.int32),
            pltpu.VMEM((CHUNK, DEG_W), jnp.float32),
            pltpu.VMEM_SHARED((N_PAD, DEG_W), jnp.float32),
        ],
        compiler_params=_sc_params,
    )(dst2d, zeros16, ones8)


def _sc_aggregate(y, src2d, dst2d, zerosD, d):
    return pl.kernel(
        _agg_body,
        out_type=jax.ShapeDtypeStruct((NC, N_PAD, d), jnp.float32),
        mesh=_vmesh,
        scratch_types=[
            pltpu.VMEM((NCH, CHUNK), jnp.int32),
            pltpu.VMEM((NCH, CHUNK), jnp.int32),
            pltpu.VMEM((NBUF, CHUNK, d), jnp.float32),
            pltpu.VMEM_SHARED((N_PAD, d), jnp.float32),
            pltpu.VMEM_SHARED((N_PAD, d), jnp.float32),
            pltpu.SemaphoreType.DMA((NBUF,)),
        ],
        compiler_params=_sc_params,
    )(y, src2d, dst2d, zerosD)


def _dinv_of(degp_ref):
    deg = degp_ref[0, :, 0:1] + degp_ref[1, :, 0:1] + 1.0
    return lax.rsqrt(jnp.maximum(deg, 1e-12))


def _scale_kernel(degp_ref, x_ref, ylo_ref, yhi_ref):
    y = x_ref[...] * _dinv_of(degp_ref)
    ylo_ref[...] = y[:, :COL]
    yhi_ref[...] = y[:, COL:]


def _mm_kernel(degp_ref, zlo_ref, zhi_ref, ylo_ref, yhi_ref, w1_ref, b1_ref,
               w2_ref, o_ref):
    dinv = _dinv_of(degp_ref)
    axlo = (zlo_ref[0] + zlo_ref[1] + ylo_ref[...]) * dinv
    axhi = (zhi_ref[0] + zhi_ref[1] + yhi_ref[...]) * dinv
    h = jnp.maximum(
        jnp.dot(axlo, w1_ref[:COL], preferred_element_type=jnp.float32)
        + jnp.dot(axhi, w1_ref[COL:], preferred_element_type=jnp.float32)
        + b1_ref[...], 0.0)
    p = jnp.dot(h, w2_ref[...], preferred_element_type=jnp.float32)
    o_ref[...] = p * dinv


def _final_kernel(degp_ref, q_ref, y2_ref, b2_ref, o_ref):
    dinv = _dinv_of(degp_ref)
    o = (q_ref[0] + q_ref[1] + y2_ref[...]) * dinv
    o40 = o[:, :OUT_DIM] + b2_ref[...]
    m = jnp.max(o40, axis=1, keepdims=True)
    ls = m + jnp.log(jnp.sum(jnp.exp(o40 - m), axis=1, keepdims=True))
    o_ref[...] = o40 - ls


def _rows(blk, d1):
    return pl.BlockSpec((blk, d1), lambda i: (i, 0))


def _rows3(n0, blk, d1):
    return pl.BlockSpec((n0, blk, d1), lambda i: (0, i, 0))


def _full(d0, d1):
    return pl.BlockSpec((d0, d1), lambda i: (0, 0))


def kernel(x, edge_index, W1, b1, W2, b2):
    f32 = jnp.float32
    src = edge_index[0]
    dst = edge_index[1]
    pad = jnp.full((E_PAD - E,), PAD_ROW, jnp.int32)
    src2d = jnp.concatenate([src, pad]).reshape(E_PAD // CHUNK, CHUNK)
    dst2d = jnp.concatenate([dst, pad]).reshape(E_PAD // CHUNK, CHUNK)
    x_pad = jnp.zeros((N_PAD, IN_DIM), f32).at[:N].set(x)
    W2p = jnp.zeros((HID_DIM, OUT_PAD), f32).at[:, :OUT_DIM].set(W2)
    zeros16 = jnp.zeros((N_PAD, DEG_W), f32)
    ones8 = jnp.ones((CHUNK, DEG_W), f32)
    zeros64 = jnp.zeros((N_PAD, COL), f32)
    zeros48 = jnp.zeros((N_PAD, OUT_PAD), f32)

    degp = _sc_degree(dst2d, zeros16, ones8)

    ylo, yhi = pl.pallas_call(
        _scale_kernel,
        grid=(16,),
        in_specs=[_rows3(NC, 640, DEG_W), _rows(640, IN_DIM)],
        out_specs=[_rows(640, COL), _rows(640, COL)],
        out_shape=[jax.ShapeDtypeStruct((N_PAD, COL), f32),
                   jax.ShapeDtypeStruct((N_PAD, COL), f32)],
    )(degp, x_pad)

    zplo = _sc_aggregate(ylo, src2d, dst2d, zeros64, COL)
    zphi = _sc_aggregate(yhi, src2d, dst2d, zeros64, COL)

    y2 = pl.pallas_call(
        _mm_kernel,
        grid=(16,),
        in_specs=[
            _rows3(NC, 640, DEG_W),
            _rows3(NC, 640, COL),
            _rows3(NC, 640, COL),
            _rows(640, COL),
            _rows(640, COL),
            _full(IN_DIM, HID_DIM),
            _full(1, HID_DIM),
            _full(HID_DIM, OUT_PAD),
        ],
        out_specs=_rows(640, OUT_PAD),
        out_shape=jax.ShapeDtypeStruct((N_PAD, OUT_PAD), f32),
    )(degp, zplo, zphi, ylo, yhi, W1, b1.reshape(1, HID_DIM), W2p)

    qp = _sc_aggregate(y2, src2d, dst2d, zeros48, OUT_PAD)

    out = pl.pallas_call(
        _final_kernel,
        grid=(25,),
        in_specs=[
            _rows3(NC, 400, DEG_W),
            _rows3(NC, 400, OUT_PAD),
            _rows(400, OUT_PAD),
            _full(1, OUT_DIM),
        ],
        out_specs=_rows(400, OUT_DIM),
        out_shape=jax.ShapeDtypeStruct((N, OUT_DIM), f32),
    )(degp, qp, y2, b2.reshape(1, OUT_DIM))
    return out


# speedup vs baseline: 1.0119x; 1.0119x over previous
import functools

import jax
import jax.numpy as jnp
from jax import lax
from jax.experimental import pallas as pl
from jax.experimental.pallas import tpu as pltpu
from jax.experimental.pallas import tpu_sc as plsc

N = 10000
E = 320000
IN_DIM = 128
HID_DIM = 1024
OUT_DIM = 40
OUT_PAD = 48

NC, NS, LANES = 2, 16, 16
NW = NC * NS
CHUNK = 64
NCH = 160
NBUF = 6
COL = 64
E_PAD = NW * NCH * CHUNK
N_PAD = 10240
STRIPE = N_PAD // NS
PAD_ROW = N
DEG_W = 8

_vmesh = plsc.VectorSubcoreMesh(core_axis_name="c", subcore_axis_name="s")
_sc_params = pltpu.CompilerParams(use_tc_tiling_on_sc=False)


def _deg_body(dst_hbm, zeros_hbm, ones_hbm, out_hbm, idx_v, ones_v, deg_sh):
    cid = lax.axis_index("c")
    sid = lax.axis_index("s")
    wid = sid * NC + cid
    row0 = sid * STRIPE
    pltpu.sync_copy(zeros_hbm.at[pl.ds(row0, STRIPE)],
                    deg_sh.at[pl.ds(row0, STRIPE)])
    pltpu.sync_copy(dst_hbm.at[pl.ds(wid * NCH, NCH)], idx_v)
    pltpu.sync_copy(ones_hbm, ones_v)
    plsc.subcore_barrier()

    @pl.loop(0, NCH)
    def _(j):
        pltpu.sync_copy(ones_v, deg_sh.at[idx_v.at[j]], add=True)

    plsc.subcore_barrier()
    pltpu.sync_copy(deg_sh.at[pl.ds(row0, STRIPE)],
                    out_hbm.at[cid, pl.ds(row0, STRIPE)])


def _agg_body(y_hbm, src_hbm, dst_hbm, zeros_hbm, out_hbm,
              idxs_v, idxd_v, rows_v, y_sh, z_sh, sems):
    cid = lax.axis_index("c")
    sid = lax.axis_index("s")
    wid = sid * NC + cid
    row0 = sid * STRIPE
    pltpu.sync_copy(y_hbm.at[pl.ds(row0, STRIPE)], y_sh.at[pl.ds(row0, STRIPE)])
    pltpu.sync_copy(zeros_hbm.at[pl.ds(row0, STRIPE)],
                    z_sh.at[pl.ds(row0, STRIPE)])
    pltpu.sync_copy(src_hbm.at[pl.ds(wid * NCH, NCH)], idxs_v)
    pltpu.sync_copy(dst_hbm.at[pl.ds(wid * NCH, NCH)], idxd_v)
    plsc.subcore_barrier()

    @pl.loop(0, NBUF)
    def _(b):
        pltpu.async_copy(y_sh.at[idxs_v.at[b]], rows_v.at[b], sems.at[b])

    @pl.loop(0, NCH - NBUF)
    def _(j):
        b = lax.rem(j, NBUF)
        pltpu.make_async_copy(y_sh.at[idxs_v.at[0]], rows_v.at[b],
                              sems.at[b]).wait()
        pltpu.sync_copy(rows_v.at[b], z_sh.at[idxd_v.at[j]], add=True)
        pltpu.async_copy(y_sh.at[idxs_v.at[j + NBUF]], rows_v.at[b], sems.at[b])

    @pl.loop(NCH - NBUF, NCH)
    def _(j):
        b = lax.rem(j, NBUF)
        pltpu.make_async_copy(y_sh.at[idxs_v.at[0]], rows_v.at[b],
                              sems.at[b]).wait()
        pltpu.sync_copy(rows_v.at[b], z_sh.at[idxd_v.at[j]], add=True)

    plsc.subcore_barrier()
    pltpu.sync_copy(z_sh.at[pl.ds(row0, STRIPE)],
                    out_hbm.at[cid, pl.ds(row0, STRIPE)])


def _sc_degree(dst2d, zeros16, ones8):
    return pl.kernel(
        _deg_body,
        out_type=jax.ShapeDtypeStruct((NC, N_PAD, DEG_W), jnp.float32),
        mesh=_vmesh,
        scratch_types=[
            pltpu.VMEM((NCH, CHUNK), jnp.int32),
            pltpu.VMEM((CHUNK, DEG_W), jnp.float32),
            pltpu.VMEM_SHARED((N_PAD, DEG_W), jnp.float32),
        ],
        compiler_params=_sc_params,
    )(dst2d, zeros16, ones8)


def _sc_aggregate(y, src2d, dst2d, zerosD, d):
    return pl.kernel(
        _agg_body,
        out_type=jax.ShapeDtypeStruct((NC, N_PAD, d), jnp.float32),
        mesh=_vmesh,
        scratch_types=[
            pltpu.VMEM((NCH, CHUNK), jnp.int32),
            pltpu.VMEM((NCH, CHUNK), jnp.int32),
            pltpu.VMEM((NBUF, CHUNK, d), jnp.float32),
            pltpu.VMEM_SHARED((N_PAD, d), jnp.float32),
            pltpu.VMEM_SHARED((N_PAD, d), jnp.float32),
            pltpu.SemaphoreType.DMA((NBUF,)),
        ],
        compiler_params=_sc_params,
    )(y, src2d, dst2d, zerosD)


def _dinv_of(degp_ref):
    deg = degp_ref[0, :, 0:1] + degp_ref[1, :, 0:1] + 1.0
    return lax.rsqrt(jnp.maximum(deg, 1e-12))


def _scale_kernel(degp_ref, x_ref, ylo_ref, yhi_ref):
    y = x_ref[...] * _dinv_of(degp_ref)
    ylo_ref[...] = y[:, :COL]
    yhi_ref[...] = y[:, COL:]


def _mm_kernel(degp_ref, zlo_ref, zhi_ref, ylo_ref, yhi_ref, w1_ref, b1_ref,
               w2_ref, o_ref):
    dinv = _dinv_of(degp_ref)
    axlo = (zlo_ref[0] + zlo_ref[1] + ylo_ref[...]) * dinv
    axhi = (zhi_ref[0] + zhi_ref[1] + yhi_ref[...]) * dinv
    h = jnp.maximum(
        jnp.dot(axlo, w1_ref[:COL], preferred_element_type=jnp.float32)
        + jnp.dot(axhi, w1_ref[COL:], preferred_element_type=jnp.float32)
        + b1_ref[...], 0.0)
    p = jnp.dot(h, w2_ref[...], preferred_element_type=jnp.float32)
    o_ref[...] = p * dinv


def _final_kernel(degp_ref, q_ref, y2_ref, b2_ref, o_ref):
    dinv = _dinv_of(degp_ref)
    o = (q_ref[0] + q_ref[1] + y2_ref[...]) * dinv
    o40 = o[:, :OUT_DIM] + b2_ref[...]
    m = jnp.max(o40, axis=1, keepdims=True)
    ls = m + jnp.log(jnp.sum(jnp.exp(o40 - m), axis=1, keepdims=True))
    o_ref[...] = o40 - ls


def _rows(blk, d1):
    return pl.BlockSpec((blk, d1), lambda i: (i, 0))


def _rows3(n0, blk, d1):
    return pl.BlockSpec((n0, blk, d1), lambda i: (0, i, 0))


def _full(d0, d1):
    return pl.BlockSpec((d0, d1), lambda i: (0, 0))


def kernel(x, edge_index, W1, b1, W2, b2):
    f32 = jnp.float32
    src = edge_index[0]
    dst = edge_index[1]
    pad = jnp.full((E_PAD - E,), PAD_ROW, jnp.int32)
    src2d = jnp.concatenate([src, pad]).reshape(E_PAD // CHUNK, CHUNK)
    dst2d = jnp.concatenate([dst, pad]).reshape(E_PAD // CHUNK, CHUNK)
    x_pad = jnp.zeros((N_PAD, IN_DIM), f32).at[:N].set(x)
    W2p = jnp.zeros((HID_DIM, OUT_PAD), f32).at[:, :OUT_DIM].set(W2)
    zeros16 = jnp.zeros((N_PAD, DEG_W), f32)
    ones8 = jnp.ones((CHUNK, DEG_W), f32)
    zeros64 = jnp.zeros((N_PAD, COL), f32)
    zeros48 = jnp.zeros((N_PAD, OUT_PAD), f32)

    degp = _sc_degree(dst2d, zeros16, ones8)

    ylo, yhi = pl.pallas_call(
        _scale_kernel,
        grid=(16,),
        in_specs=[_rows3(NC, 640, DEG_W), _rows(640, IN_DIM)],
        out_specs=[_rows(640, COL), _rows(640, COL)],
        out_shape=[jax.ShapeDtypeStruct((N_PAD, COL), f32),
                   jax.ShapeDtypeStruct((N_PAD, COL), f32)],
    )(degp, x_pad)

    zplo = _sc_aggregate(ylo, src2d, dst2d, zeros64, COL)
    zphi = _sc_aggregate(yhi, src2d, dst2d, zeros64, COL)

    y2 = pl.pallas_call(
        _mm_kernel,
        grid=(16,),
        in_specs=[
            _rows3(NC, 640, DEG_W),
            _rows3(NC, 640, COL),
            _rows3(NC, 640, COL),
            _rows(640, COL),
            _rows(640, COL),
            _full(IN_DIM, HID_DIM),
            _full(1, HID_DIM),
            _full(HID_DIM, OUT_PAD),
        ],
        out_specs=_rows(640, OUT_PAD),
        out_shape=jax.ShapeDtypeStruct((N_PAD, OUT_PAD), f32),
    )(degp, zplo, zphi, ylo, yhi, W1, b1.reshape(1, HID_DIM), W2p)

    qp = _sc_aggregate(y2, src2d, dst2d, zeros48, OUT_PAD)

    out = pl.pallas_call(
        _final_kernel,
        grid=(25,),
        in_specs=[
            _rows3(NC, 400, DEG_W),
            _rows3(NC, 400, OUT_PAD),
            _rows(400, OUT_PAD),
            _full(1, OUT_DIM),
        ],
        out_specs=_rows(400, OUT_DIM),
        out_shape=jax.ShapeDtypeStruct((N, OUT_DIM), f32),
    )(degp, qp, y2, b2.reshape(1, OUT_DIM))
    return out


# speedup vs baseline: 1.0633x; 1.0509x over previous
import functools

import jax
import jax.numpy as jnp
from jax import lax
from jax.experimental import pallas as pl
from jax.experimental.pallas import tpu as pltpu
from jax.experimental.pallas import tpu_sc as plsc

N = 10000
E = 320000
IN_DIM = 128
HID_DIM = 1024
OUT_DIM = 40
OUT_PAD = 48

NC, NS, LANES = 2, 16, 16
NW = NC * NS
CHUNK = 64
NCH = 160
NBUF = 6
COL = 64
E_PAD = NW * NCH * CHUNK
N_PAD = 10240
STRIPE = N_PAD // NS
PAD_ROW = N
DEG_W = 8

_vmesh = plsc.VectorSubcoreMesh(core_axis_name="c", subcore_axis_name="s")
_sc_params = pltpu.CompilerParams(use_tc_tiling_on_sc=False)


def _deg_body(dst_hbm, zeros_hbm, ones_hbm, out_hbm, idx_v, ones_v, deg_sh):
    cid = lax.axis_index("c")
    sid = lax.axis_index("s")
    wid = sid * NC + cid
    row0 = sid * STRIPE
    pltpu.sync_copy(zeros_hbm.at[pl.ds(row0, STRIPE)],
                    deg_sh.at[pl.ds(row0, STRIPE)])
    pltpu.sync_copy(dst_hbm.at[pl.ds(wid * NCH, NCH)], idx_v)
    pltpu.sync_copy(ones_hbm, ones_v)
    plsc.subcore_barrier()

    @pl.loop(0, NCH)
    def _(j):
        pltpu.sync_copy(ones_v, deg_sh.at[idx_v.at[j]], add=True)

    plsc.subcore_barrier()
    pltpu.sync_copy(deg_sh.at[pl.ds(row0, STRIPE)],
                    out_hbm.at[cid, pl.ds(row0, STRIPE)])


def _agg_body(y_hbm, src_hbm, dst_hbm, zeros_hbm, out_hbm,
              idxs_v, idxd_v, rows_v, y_sh, z_sh, sems):
    cid = lax.axis_index("c")
    sid = lax.axis_index("s")
    wid = sid * NC + cid
    row0 = sid * STRIPE
    pltpu.sync_copy(y_hbm.at[pl.ds(row0, STRIPE)], y_sh.at[pl.ds(row0, STRIPE)])
    pltpu.sync_copy(zeros_hbm.at[pl.ds(row0, STRIPE)],
                    z_sh.at[pl.ds(row0, STRIPE)])
    pltpu.sync_copy(src_hbm.at[pl.ds(wid * NCH, NCH)], idxs_v)
    pltpu.sync_copy(dst_hbm.at[pl.ds(wid * NCH, NCH)], idxd_v)
    plsc.subcore_barrier()

    @pl.loop(0, NBUF)
    def _(b):
        pltpu.async_copy(y_sh.at[idxs_v.at[b]], rows_v.at[b], sems.at[b])

    @pl.loop(0, NCH - NBUF)
    def _(j):
        b = lax.rem(j, NBUF)
        pltpu.make_async_copy(y_sh.at[idxs_v.at[0]], rows_v.at[b],
                              sems.at[b]).wait()
        pltpu.sync_copy(rows_v.at[b], z_sh.at[idxd_v.at[j]], add=True)
        pltpu.async_copy(y_sh.at[idxs_v.at[j + NBUF]], rows_v.at[b], sems.at[b])

    @pl.loop(NCH - NBUF, NCH)
    def _(j):
        b = lax.rem(j, NBUF)
        pltpu.make_async_copy(y_sh.at[idxs_v.at[0]], rows_v.at[b],
                              sems.at[b]).wait()
        pltpu.sync_copy(rows_v.at[b], z_sh.at[idxd_v.at[j]], add=True)

    plsc.subcore_barrier()
    pltpu.sync_copy(z_sh.at[pl.ds(row0, STRIPE)],
                    out_hbm.at[cid, pl.ds(row0, STRIPE)])


def _agg2pass_body(ylo_hbm, yhi_hbm, src_hbm, dst_hbm, zeros_hbm,
                   outlo_hbm, outhi_hbm, idxs_v, idxd_v, rows_v, y_sh, z_sh,
                   sems):
    cid = lax.axis_index("c")
    sid = lax.axis_index("s")
    wid = sid * NC + cid
    row0 = sid * STRIPE
    pltpu.sync_copy(src_hbm.at[pl.ds(wid * NCH, NCH)], idxs_v)
    pltpu.sync_copy(dst_hbm.at[pl.ds(wid * NCH, NCH)], idxd_v)

    for y_hbm, out_hbm in ((ylo_hbm, outlo_hbm), (yhi_hbm, outhi_hbm)):
        pltpu.sync_copy(y_hbm.at[pl.ds(row0, STRIPE)],
                        y_sh.at[pl.ds(row0, STRIPE)])
        pltpu.sync_copy(zeros_hbm.at[pl.ds(row0, STRIPE)],
                        z_sh.at[pl.ds(row0, STRIPE)])
        plsc.subcore_barrier()

        @pl.loop(0, NBUF)
        def _(b):
            pltpu.async_copy(y_sh.at[idxs_v.at[b]], rows_v.at[b], sems.at[b])

        @pl.loop(0, NCH - NBUF)
        def _(j):
            b = lax.rem(j, NBUF)
            pltpu.make_async_copy(y_sh.at[idxs_v.at[0]], rows_v.at[b],
                                  sems.at[b]).wait()
            pltpu.sync_copy(rows_v.at[b], z_sh.at[idxd_v.at[j]], add=True)
            pltpu.async_copy(y_sh.at[idxs_v.at[j + NBUF]], rows_v.at[b],
                             sems.at[b])

        @pl.loop(NCH - NBUF, NCH)
        def _(j):
            b = lax.rem(j, NBUF)
            pltpu.make_async_copy(y_sh.at[idxs_v.at[0]], rows_v.at[b],
                                  sems.at[b]).wait()
            pltpu.sync_copy(rows_v.at[b], z_sh.at[idxd_v.at[j]], add=True)

        plsc.subcore_barrier()
        pltpu.sync_copy(z_sh.at[pl.ds(row0, STRIPE)],
                        out_hbm.at[cid, pl.ds(row0, STRIPE)])
        plsc.subcore_barrier()


def _sc_aggregate_2pass(ylo, yhi, src2d, dst2d, zerosD):
    return pl.kernel(
        _agg2pass_body,
        out_type=[jax.ShapeDtypeStruct((NC, N_PAD, COL), jnp.float32),
                  jax.ShapeDtypeStruct((NC, N_PAD, COL), jnp.float32)],
        mesh=_vmesh,
        scratch_types=[
            pltpu.VMEM((NCH, CHUNK), jnp.int32),
            pltpu.VMEM((NCH, CHUNK), jnp.int32),
            pltpu.VMEM((NBUF, CHUNK, COL), jnp.float32),
            pltpu.VMEM_SHARED((N_PAD, COL), jnp.float32),
            pltpu.VMEM_SHARED((N_PAD, COL), jnp.float32),
            pltpu.SemaphoreType.DMA((NBUF,)),
        ],
        compiler_params=_sc_params,
    )(ylo, yhi, src2d, dst2d, zerosD)


def _sc_degree(dst2d, zeros16, ones8):
    return pl.kernel(
        _deg_body,
        out_type=jax.ShapeDtypeStruct((NC, N_PAD, DEG_W), jnp.float32),
        mesh=_vmesh,
        scratch_types=[
            pltpu.VMEM((NCH, CHUNK), jnp.int32),
            pltpu.VMEM((CHUNK, DEG_W), jnp.float32),
            pltpu.VMEM_SHARED((N_PAD, DEG_W), jnp.float32),
        ],
        compiler_params=_sc_params,
    )(dst2d, zeros16, ones8)


def _sc_aggregate(y, src2d, dst2d, zerosD, d):
    return pl.kernel(
        _agg_body,
        out_type=jax.ShapeDtypeStruct((NC, N_PAD, d), jnp.float32),
        mesh=_vmesh,
        scratch_types=[
            pltpu.VMEM((NCH, CHUNK), jnp.int32),
            pltpu.VMEM((NCH, CHUNK), jnp.int32),
            pltpu.VMEM((NBUF, CHUNK, d), jnp.float32),
            pltpu.VMEM_SHARED((N_PAD, d), jnp.float32),
            pltpu.VMEM_SHARED((N_PAD, d), jnp.float32),
            pltpu.SemaphoreType.DMA((NBUF,)),
        ],
        compiler_params=_sc_params,
    )(y, src2d, dst2d, zerosD)


def _dinv_of(degp_ref):
    deg = degp_ref[0, :, 0:1] + degp_ref[1, :, 0:1] + 1.0
    return lax.rsqrt(jnp.maximum(deg, 1e-12))


def _scale_kernel(degp_ref, x_ref, ylo_ref, yhi_ref):
    y = x_ref[...] * _dinv_of(degp_ref)
    ylo_ref[...] = y[:, :COL]
    yhi_ref[...] = y[:, COL:]


def _mm_kernel(degp_ref, zlo_ref, zhi_ref, ylo_ref, yhi_ref, w1_ref, b1_ref,
               w2_ref, o_ref):
    dinv = _dinv_of(degp_ref)
    axlo = (zlo_ref[0] + zlo_ref[1] + ylo_ref[...]) * dinv
    axhi = (zhi_ref[0] + zhi_ref[1] + yhi_ref[...]) * dinv
    bf = jnp.bfloat16
    h = jnp.maximum(
        jnp.dot(axlo.astype(bf), w1_ref[:COL].astype(bf),
                preferred_element_type=jnp.float32)
        + jnp.dot(axhi.astype(bf), w1_ref[COL:].astype(bf),
                  preferred_element_type=jnp.float32)
        + b1_ref[...], 0.0)
    p = jnp.dot(h.astype(bf), w2_ref[...].astype(bf),
                preferred_element_type=jnp.float32)
    o_ref[...] = p * dinv


def _final_kernel(degp_ref, q_ref, y2_ref, b2_ref, o_ref):
    dinv = _dinv_of(degp_ref)
    o = (q_ref[0] + q_ref[1] + y2_ref[...]) * dinv
    o40 = o[:, :OUT_DIM] + b2_ref[...]
    m = jnp.max(o40, axis=1, keepdims=True)
    ls = m + jnp.log(jnp.sum(jnp.exp(o40 - m), axis=1, keepdims=True))
    o_ref[...] = o40 - ls


def _rows(blk, d1):
    return pl.BlockSpec((blk, d1), lambda i: (i, 0))


def _rows3(n0, blk, d1):
    return pl.BlockSpec((n0, blk, d1), lambda i: (0, i, 0))


def _full(d0, d1):
    return pl.BlockSpec((d0, d1), lambda i: (0, 0))


def kernel(x, edge_index, W1, b1, W2, b2):
    f32 = jnp.float32
    src = edge_index[0]
    dst = edge_index[1]
    pad = jnp.full((E_PAD - E,), PAD_ROW, jnp.int32)
    src2d = jnp.concatenate([src, pad]).reshape(E_PAD // CHUNK, CHUNK)
    dst2d = jnp.concatenate([dst, pad]).reshape(E_PAD // CHUNK, CHUNK)
    x_pad = jnp.zeros((N_PAD, IN_DIM), f32).at[:N].set(x)
    W2p = jnp.zeros((HID_DIM, OUT_PAD), f32).at[:, :OUT_DIM].set(W2)
    zeros16 = jnp.zeros((N_PAD, DEG_W), f32)
    ones8 = jnp.ones((CHUNK, DEG_W), f32)
    zeros64 = jnp.zeros((N_PAD, COL), f32)
    zeros48 = jnp.zeros((N_PAD, OUT_PAD), f32)

    degp = _sc_degree(dst2d, zeros16, ones8)

    ylo, yhi = pl.pallas_call(
        _scale_kernel,
        grid=(4,),
        in_specs=[_rows3(NC, 2560, DEG_W), _rows(2560, IN_DIM)],
        out_specs=[_rows(2560, COL), _rows(2560, COL)],
        out_shape=[jax.ShapeDtypeStruct((N_PAD, COL), f32),
                   jax.ShapeDtypeStruct((N_PAD, COL), f32)],
    )(degp, x_pad)

    zplo, zphi = _sc_aggregate_2pass(ylo, yhi, src2d, dst2d, zeros64)

    y2 = pl.pallas_call(
        _mm_kernel,
        grid=(8,),
        in_specs=[
            _rows3(NC, 1280, DEG_W),
            _rows3(NC, 1280, COL),
            _rows3(NC, 1280, COL),
            _rows(1280, COL),
            _rows(1280, COL),
            _full(IN_DIM, HID_DIM),
            _full(1, HID_DIM),
            _full(HID_DIM, OUT_PAD),
        ],
        out_specs=_rows(1280, OUT_PAD),
        out_shape=jax.ShapeDtypeStruct((N_PAD, OUT_PAD), f32),
    )(degp, zplo, zphi, ylo, yhi, W1, b1.reshape(1, HID_DIM), W2p)

    qp = _sc_aggregate(y2, src2d, dst2d, zeros48, OUT_PAD)

    out = pl.pallas_call(
        _final_kernel,
        grid=(5,),
        in_specs=[
            _rows3(NC, 2000, DEG_W),
            _rows3(NC, 2000, OUT_PAD),
            _rows(2000, OUT_PAD),
            _full(1, OUT_DIM),
        ],
        out_specs=_rows(2000, OUT_DIM),
        out_shape=jax.ShapeDtypeStruct((N, OUT_DIM), f32),
    )(degp, qp, y2, b2.reshape(1, OUT_DIM))
    return out


# speedup vs baseline: 1.1540x; 1.0853x over previous
import functools

import jax
import jax.numpy as jnp
from jax import lax
from jax.experimental import pallas as pl
from jax.experimental.pallas import tpu as pltpu
from jax.experimental.pallas import tpu_sc as plsc

N = 10000
E = 320000
IN_DIM = 128
HID_DIM = 1024
OUT_DIM = 40
OUT_PAD = 48

NC, NS, LANES = 2, 16, 16
NW = NC * NS
CHUNK = 64
NCH = 160
NBUF = 6
COL = 64
E_PAD = NW * NCH * CHUNK
N_PAD = 10240
STRIPE = N_PAD // NS
PAD_ROW = N
DEG_W = 8

_vmesh = plsc.VectorSubcoreMesh(core_axis_name="c", subcore_axis_name="s")
_sc_params = pltpu.CompilerParams(use_tc_tiling_on_sc=False)


def _deg_body(dst_hbm, zeros_hbm, ones_hbm, out_hbm, idx_v, ones_v, deg_sh,
              ssems):
    cid = lax.axis_index("c")
    sid = lax.axis_index("s")
    wid = sid * NC + cid
    row0 = sid * STRIPE
    pltpu.sync_copy(zeros_hbm.at[pl.ds(row0, STRIPE)],
                    deg_sh.at[pl.ds(row0, STRIPE)])
    pltpu.sync_copy(dst_hbm.at[pl.ds(wid * NCH, NCH)], idx_v)
    pltpu.sync_copy(ones_hbm, ones_v)
    plsc.subcore_barrier()

    @pl.loop(0, NCH)
    def _(j):
        b = lax.rem(j, NBUF)

        @pl.when(j >= NBUF)
        def _():
            pltpu.make_async_copy(ones_v, deg_sh.at[idx_v.at[0]],
                                  ssems.at[b]).wait()
        pltpu.async_copy(ones_v, deg_sh.at[idx_v.at[j]], ssems.at[b], add=True)

    @pl.loop(0, NBUF)
    def _(b):
        pltpu.make_async_copy(ones_v, deg_sh.at[idx_v.at[0]],
                              ssems.at[b]).wait()

    plsc.subcore_barrier()
    pltpu.sync_copy(deg_sh.at[pl.ds(row0, STRIPE)],
                    out_hbm.at[cid, pl.ds(row0, STRIPE)])


def _agg_body(y_hbm, src_hbm, dst_hbm, zeros_hbm, out_hbm,
              idxs_v, idxd_v, rows_v, y_sh, z_sh, sems):
    cid = lax.axis_index("c")
    sid = lax.axis_index("s")
    wid = sid * NC + cid
    row0 = sid * STRIPE
    pltpu.sync_copy(y_hbm.at[pl.ds(row0, STRIPE)], y_sh.at[pl.ds(row0, STRIPE)])
    pltpu.sync_copy(zeros_hbm.at[pl.ds(row0, STRIPE)],
                    z_sh.at[pl.ds(row0, STRIPE)])
    pltpu.sync_copy(src_hbm.at[pl.ds(wid * NCH, NCH)], idxs_v)
    pltpu.sync_copy(dst_hbm.at[pl.ds(wid * NCH, NCH)], idxd_v)
    plsc.subcore_barrier()

    @pl.loop(0, NBUF)
    def _(b):
        pltpu.async_copy(y_sh.at[idxs_v.at[b]], rows_v.at[b], sems.at[b])

    @pl.loop(0, NCH - NBUF)
    def _(j):
        b = lax.rem(j, NBUF)
        pltpu.make_async_copy(y_sh.at[idxs_v.at[0]], rows_v.at[b],
                              sems.at[b]).wait()
        pltpu.sync_copy(rows_v.at[b], z_sh.at[idxd_v.at[j]], add=True)
        pltpu.async_copy(y_sh.at[idxs_v.at[j + NBUF]], rows_v.at[b], sems.at[b])

    @pl.loop(NCH - NBUF, NCH)
    def _(j):
        b = lax.rem(j, NBUF)
        pltpu.make_async_copy(y_sh.at[idxs_v.at[0]], rows_v.at[b],
                              sems.at[b]).wait()
        pltpu.sync_copy(rows_v.at[b], z_sh.at[idxd_v.at[j]], add=True)

    plsc.subcore_barrier()
    pltpu.sync_copy(z_sh.at[pl.ds(row0, STRIPE)],
                    out_hbm.at[cid, pl.ds(row0, STRIPE)])


def _agg2pass_body(ylo_hbm, yhi_hbm, src_hbm, dst_hbm, zeros_hbm,
                   outlo_hbm, outhi_hbm, idxs_v, idxd_v, rows_v, y_sh, z_sh,
                   sems):
    cid = lax.axis_index("c")
    sid = lax.axis_index("s")
    wid = sid * NC + cid
    row0 = sid * STRIPE
    pltpu.sync_copy(src_hbm.at[pl.ds(wid * NCH, NCH)], idxs_v)
    pltpu.sync_copy(dst_hbm.at[pl.ds(wid * NCH, NCH)], idxd_v)

    for y_hbm, out_hbm in ((ylo_hbm, outlo_hbm), (yhi_hbm, outhi_hbm)):
        pltpu.sync_copy(y_hbm.at[pl.ds(row0, STRIPE)],
                        y_sh.at[pl.ds(row0, STRIPE)])
        pltpu.sync_copy(zeros_hbm.at[pl.ds(row0, STRIPE)],
                        z_sh.at[pl.ds(row0, STRIPE)])
        plsc.subcore_barrier()

        @pl.loop(0, NBUF)
        def _(b):
            pltpu.async_copy(y_sh.at[idxs_v.at[b]], rows_v.at[b], sems.at[b])

        @pl.loop(0, NCH - NBUF)
        def _(j):
            b = lax.rem(j, NBUF)
            pltpu.make_async_copy(y_sh.at[idxs_v.at[0]], rows_v.at[b],
                                  sems.at[b]).wait()
            pltpu.sync_copy(rows_v.at[b], z_sh.at[idxd_v.at[j]], add=True)
            pltpu.async_copy(y_sh.at[idxs_v.at[j + NBUF]], rows_v.at[b],
                             sems.at[b])

        @pl.loop(NCH - NBUF, NCH)
        def _(j):
            b = lax.rem(j, NBUF)
            pltpu.make_async_copy(y_sh.at[idxs_v.at[0]], rows_v.at[b],
                                  sems.at[b]).wait()
            pltpu.sync_copy(rows_v.at[b], z_sh.at[idxd_v.at[j]], add=True)

        plsc.subcore_barrier()
        pltpu.sync_copy(z_sh.at[pl.ds(row0, STRIPE)],
                        out_hbm.at[cid, pl.ds(row0, STRIPE)])
        plsc.subcore_barrier()


def _sc_aggregate_2pass(ylo, yhi, src2d, dst2d, zerosD):
    return pl.kernel(
        _agg2pass_body,
        out_type=[jax.ShapeDtypeStruct((NC, N_PAD, COL), jnp.float32),
                  jax.ShapeDtypeStruct((NC, N_PAD, COL), jnp.float32)],
        mesh=_vmesh,
        scratch_types=[
            pltpu.VMEM((NCH, CHUNK), jnp.int32),
            pltpu.VMEM((NCH, CHUNK), jnp.int32),
            pltpu.VMEM((NBUF, CHUNK, COL), jnp.float32),
            pltpu.VMEM_SHARED((N_PAD, COL), jnp.float32),
            pltpu.VMEM_SHARED((N_PAD, COL), jnp.float32),
            pltpu.SemaphoreType.DMA((NBUF,)),
        ],
        compiler_params=_sc_params,
    )(ylo, yhi, src2d, dst2d, zerosD)


def _sc_degree(dst2d, zeros16, ones8):
    return pl.kernel(
        _deg_body,
        out_type=jax.ShapeDtypeStruct((NC, N_PAD, DEG_W), jnp.float32),
        mesh=_vmesh,
        scratch_types=[
            pltpu.VMEM((NCH, CHUNK), jnp.int32),
            pltpu.VMEM((CHUNK, DEG_W), jnp.float32),
            pltpu.VMEM_SHARED((N_PAD, DEG_W), jnp.float32),
            pltpu.SemaphoreType.DMA((NBUF,)),
        ],
        compiler_params=_sc_params,
    )(dst2d, zeros16, ones8)


def _sc_aggregate(y, src2d, dst2d, zerosD, d):
    return pl.kernel(
        _agg_body,
        out_type=jax.ShapeDtypeStruct((NC, N_PAD, d), jnp.float32),
        mesh=_vmesh,
        scratch_types=[
            pltpu.VMEM((NCH, CHUNK), jnp.int32),
            pltpu.VMEM((NCH, CHUNK), jnp.int32),
            pltpu.VMEM((NBUF, CHUNK, d), jnp.float32),
            pltpu.VMEM_SHARED((N_PAD, d), jnp.float32),
            pltpu.VMEM_SHARED((N_PAD, d), jnp.float32),
            pltpu.SemaphoreType.DMA((NBUF,)),
        ],
        compiler_params=_sc_params,
    )(y, src2d, dst2d, zerosD)


def _dinv_of(degp_ref):
    deg = degp_ref[0, :, 0:1] + degp_ref[1, :, 0:1] + 1.0
    return lax.rsqrt(jnp.maximum(deg, 1e-12))


def _scale_kernel(degp_ref, x_ref, ylo_ref, yhi_ref):
    y = x_ref[...] * _dinv_of(degp_ref)
    ylo_ref[...] = y[:, :COL]
    yhi_ref[...] = y[:, COL:]


def _mm_kernel(degp_ref, zlo_ref, zhi_ref, ylo_ref, yhi_ref, w1_ref, b1_ref,
               w2_ref, o_ref):
    dinv = _dinv_of(degp_ref)
    axlo = (zlo_ref[0] + zlo_ref[1] + ylo_ref[...]) * dinv
    axhi = (zhi_ref[0] + zhi_ref[1] + yhi_ref[...]) * dinv
    bf = jnp.bfloat16
    h = jnp.maximum(
        jnp.dot(axlo.astype(bf), w1_ref[:COL].astype(bf),
                preferred_element_type=jnp.float32)
        + jnp.dot(axhi.astype(bf), w1_ref[COL:].astype(bf),
                  preferred_element_type=jnp.float32)
        + b1_ref[...], 0.0)
    p = jnp.dot(h.astype(bf), w2_ref[...].astype(bf),
                preferred_element_type=jnp.float32)
    o_ref[...] = p * dinv


def _final_kernel(degp_ref, q_ref, y2_ref, b2_ref, o_ref):
    dinv = _dinv_of(degp_ref)
    o = (q_ref[0] + q_ref[1] + y2_ref[...]) * dinv
    o40 = o[:, :OUT_DIM] + b2_ref[...]
    m = jnp.max(o40, axis=1, keepdims=True)
    ls = m + jnp.log(jnp.sum(jnp.exp(o40 - m), axis=1, keepdims=True))
    o_ref[...] = o40 - ls


def _rows(blk, d1):
    return pl.BlockSpec((blk, d1), lambda i: (i, 0))


def _rows3(n0, blk, d1):
    return pl.BlockSpec((n0, blk, d1), lambda i: (0, i, 0))


def _full(d0, d1):
    return pl.BlockSpec((d0, d1), lambda i: (0, 0))


def kernel(x, edge_index, W1, b1, W2, b2):
    f32 = jnp.float32
    src = edge_index[0]
    dst = edge_index[1]
    pad = jnp.full((E_PAD - E,), PAD_ROW, jnp.int32)
    src2d = jnp.concatenate([src, pad]).reshape(E_PAD // CHUNK, CHUNK)
    dst2d = jnp.concatenate([dst, pad]).reshape(E_PAD // CHUNK, CHUNK)
    x_pad = jnp.zeros((N_PAD, IN_DIM), f32).at[:N].set(x)
    W2p = jnp.zeros((HID_DIM, OUT_PAD), f32).at[:, :OUT_DIM].set(W2)
    zeros16 = jnp.zeros((N_PAD, DEG_W), f32)
    ones8 = jnp.ones((CHUNK, DEG_W), f32)
    zeros64 = jnp.zeros((N_PAD, COL), f32)
    zeros48 = jnp.zeros((N_PAD, OUT_PAD), f32)

    degp = _sc_degree(dst2d, zeros16, ones8)

    ylo, yhi = pl.pallas_call(
        _scale_kernel,
        grid=(4,),
        in_specs=[_rows3(NC, 2560, DEG_W), _rows(2560, IN_DIM)],
        out_specs=[_rows(2560, COL), _rows(2560, COL)],
        out_shape=[jax.ShapeDtypeStruct((N_PAD, COL), f32),
                   jax.ShapeDtypeStruct((N_PAD, COL), f32)],
    )(degp, x_pad)

    zplo, zphi = _sc_aggregate_2pass(ylo, yhi, src2d, dst2d, zeros64)

    y2 = pl.pallas_call(
        _mm_kernel,
        grid=(4,),
        in_specs=[
            _rows3(NC, 2560, DEG_W),
            _rows3(NC, 2560, COL),
            _rows3(NC, 2560, COL),
            _rows(2560, COL),
            _rows(2560, COL),
            _full(IN_DIM, HID_DIM),
            _full(1, HID_DIM),
            _full(HID_DIM, OUT_PAD),
        ],
        out_specs=_rows(2560, OUT_PAD),
        out_shape=jax.ShapeDtypeStruct((N_PAD, OUT_PAD), f32),
    )(degp, zplo, zphi, ylo, yhi, W1, b1.reshape(1, HID_DIM), W2p)

    qp = _sc_aggregate(y2, src2d, dst2d, zeros48, OUT_PAD)

    out = pl.pallas_call(
        _final_kernel,
        grid=(5,),
        in_specs=[
            _rows3(NC, 2000, DEG_W),
            _rows3(NC, 2000, OUT_PAD),
            _rows(2000, OUT_PAD),
            _full(1, OUT_DIM),
        ],
        out_specs=_rows(2000, OUT_DIM),
        out_shape=jax.ShapeDtypeStruct((N, OUT_DIM), f32),
    )(degp, qp, y2, b2.reshape(1, OUT_DIM))
    return out
